# baseline (speedup 1.0000x reference)
import functools

import jax
import jax.numpy as jnp
from jax import lax
from jax.experimental import pallas as pl
from jax.experimental.pallas import tpu as pltpu
from jax.experimental.pallas import tpu_sc as plsc

_NC, _NS, _L = 2, 16, 16
_NW = _NC * _NS
_B, _S, _D = 4, 8192, 1024
_SB = _S // _NW
_C = 16
_NCH = _SB // _C


def _sc_body(x_hbm, t_hbm, o_hbm, tbuf, xb0, xb1, xb2, xb3,
             ld0, ld1, ld2, ld3, st0, st1, st2, st3):
    cid = lax.axis_index("c")
    sid = lax.axis_index("s")
    wid = sid * _NC + cid
    s0 = wid * _SB

    xbs = (xb0, xb1, xb2, xb3)
    lds = (ld0, ld1, ld2, ld3)
    sts = (st0, st1, st2, st3)

    def chunk(ci, carry):
        srow = s0 + ci * _C
        rows = pl.ds(srow, _C)
        handles = []
        for b in range(4):
            @pl.when(ci > 0)
            def _(b=b):
                pltpu.make_async_copy(
                    xbs[b], o_hbm.at[b, pl.ds(0, _C), :], sts[b]).wait()
            handles.append(
                pltpu.async_copy(x_hbm.at[b, rows, :], xbs[b], lds[b]))
        pltpu.sync_copy(t_hbm.at[rows, :], tbuf)
        for b in range(4):
            handles[b].wait()

        def tile_body(tc_i, c):
            col0 = tc_i * 128
            for r in range(_C):
                for cc in range(128 // _L):
                    sl = pl.ds(col0 + cc * _L, _L)
                    v = tbuf[r, sl]
                    for b in range(4):
                        plsc.addupdate(xbs[b].at[r, sl], v)
            return c

        lax.fori_loop(0, _D // 128, tile_body, None)
        for b in range(4):
            pltpu.async_copy(xbs[b], o_hbm.at[b, rows, :], sts[b])
        return carry

    lax.fori_loop(0, _NCH, chunk, None)
    for b in range(4):
        pltpu.make_async_copy(
            xbs[b], o_hbm.at[b, pl.ds(0, _C), :], sts[b]).wait()


@jax.jit
def _sc_add(x, t):
    mesh = plsc.VectorSubcoreMesh(
        core_axis_name="c", subcore_axis_name="s",
        num_cores=_NC, num_subcores=_NS)
    f = pl.kernel(
        _sc_body,
        out_type=jax.ShapeDtypeStruct((_B, _S, _D), jnp.float32),
        mesh=mesh,
        scratch_types=(
            [pltpu.VMEM((_C, _D), jnp.float32)] * 5
            + [pltpu.SemaphoreType.DMA] * 8
        ),
        compiler_params=pltpu.CompilerParams(use_tc_tiling_on_sc=True),
    )
    return f(x, t)


def _tc_add_body(x_ref, t_ref, o_ref):
    o_ref[...] = x_ref[...] + t_ref[...]


def _tc_add(input_embeddings, emb_table):
    B, S, D = input_embeddings.shape
    BS = 2048
    grid = (S // BS, B)
    return pl.pallas_call(
        _tc_add_body,
        grid=grid,
        in_specs=[
            pl.BlockSpec((1, BS, D), lambda s, b: (b, s, 0)),
            pl.BlockSpec((BS, D), lambda s, b: (s, 0)),
        ],
        out_specs=pl.BlockSpec((1, BS, D), lambda s, b: (b, s, 0)),
        out_shape=jax.ShapeDtypeStruct((B, S, D), jnp.float32),
    )(input_embeddings, emb_table)


def kernel(input_embeddings, emb_table):
    return (_tc_add(input_embeddings, emb_table),
            _sc_add(input_embeddings, emb_table))


# speedup vs baseline: 2.5895x; 2.5895x over previous
import jax
import jax.numpy as jnp
from jax.experimental import pallas as pl


def _add_body(x_ref, t_ref, o_ref):
    o_ref[...] = x_ref[...] + t_ref[...]


def kernel(input_embeddings, emb_table):
    B, S, D = input_embeddings.shape
    BB, BS = 2, 1024
    grid = (S // BS, B // BB)
    return pl.pallas_call(
        _add_body,
        grid=grid,
        in_specs=[
            pl.BlockSpec((BB, BS, D), lambda s, b: (b, s, 0)),
            pl.BlockSpec((BS, D), lambda s, b: (s, 0)),
        ],
        out_specs=pl.BlockSpec((BB, BS, D), lambda s, b: (b, s, 0)),
        out_shape=jax.ShapeDtypeStruct((B, S, D), jnp.float32),
    )(input_embeddings, emb_table)
